# baseline (speedup 1.0000x reference)
import functools

import jax
import jax.numpy as jnp
from jax import lax
from jax.experimental import pallas as pl
from jax.experimental.pallas import tpu as pltpu
from jax.experimental.pallas import tpu_sc as plsc

N_NODES = 10000
N_EDGES = 320000
D = 128
CHUNK = 128
SUPER = 10
NC = 2
NS = 16
NW = NC * NS
N_CHUNKS = N_EDGES // CHUNK
N_SUPER = N_CHUNKS // SUPER
N_PAD = 10112
ROWS_PER_TILE = N_PAD // NS
NBUF = 2


def _sc_body(src_hbm, dst_hbm, w_hbm, x_hbm, out_hbm,
             src_v, dst_v, w_v, rows0, rows1, acc_sh,
             gsem0, gsem1, ssem0, ssem1):
    cid = lax.axis_index("c")
    sid = lax.axis_index("s")
    wid = sid * NC + cid
    rows = (rows0, rows1)
    gsems = (gsem0, gsem1)
    ssems = (ssem0, ssem1)

    def _zrow(i, _):
        for c in range(D // 16):
            rows0[i, pl.ds(c * 16, 16)] = jnp.zeros((16,), jnp.float32)
        return 0
    lax.fori_loop(0, CHUNK, _zrow, 0)
    row0 = sid * ROWS_PER_TILE
    for b in range(ROWS_PER_TILE // CHUNK):
        pltpu.sync_copy(rows0, acc_sh.at[pl.ds(row0 + b * CHUNK, CHUNK)])
    plsc.subcore_barrier()

    n_my = (N_SUPER - wid + NW - 1) // NW

    def _super(k, _):
        s = wid + k * NW
        pltpu.sync_copy(src_hbm.at[s], src_v)
        pltpu.sync_copy(dst_hbm.at[s], dst_v)
        pltpu.sync_copy(w_hbm.at[s], w_v)

        AHEAD = 1

        ghandles, shandles = {}, {}

        def _gather(j):
            ghandles[j] = pltpu.async_copy(
                x_hbm.at[src_v.at[j]], rows[j % NBUF], gsems[j % NBUF])

        for j in range(AHEAD):
            _gather(j)

        for j in range(SUPER):
            b = j % NBUF
            buf = rows[b]
            ghandles.pop(j).wait()

            def _scale(g, _):
                wgrp = w_v[j, pl.ds(g * 16, 16)]
                for lane in range(16):
                    ws = wgrp[lane]
                    row = g * 16 + lane
                    for c in range(D // 16):
                        seg = buf[row, pl.ds(c * 16, 16)]
                        buf[row, pl.ds(c * 16, 16)] = seg * ws
                return 0
            lax.fori_loop(0, CHUNK // 16, _scale, 0)

            shandles[j] = pltpu.async_copy(
                buf, acc_sh.at[dst_v.at[j]], ssems[b], add=True)
            if j >= AHEAD:
                shandles.pop(j - AHEAD).wait()
            if j + AHEAD < SUPER:
                _gather(j + AHEAD)

        for j in range(SUPER - AHEAD, SUPER):
            shandles.pop(j).wait()
        return 0
    lax.fori_loop(0, n_my, _super, 0)

    plsc.subcore_barrier()
    pltpu.sync_copy(acc_sh.at[pl.ds(row0, ROWS_PER_TILE)],
                    out_hbm.at[cid, pl.ds(row0, ROWS_PER_TILE)])


_sc_call = functools.partial(
    pl.kernel,
    mesh=plsc.VectorSubcoreMesh(core_axis_name="c", subcore_axis_name="s"),
    out_type=jax.ShapeDtypeStruct((NC, N_PAD, D), jnp.float32),
    scratch_types=[
        pltpu.VMEM((SUPER, CHUNK), jnp.int32),
        pltpu.VMEM((SUPER, CHUNK), jnp.int32),
        pltpu.VMEM((SUPER, CHUNK), jnp.float32),
        pltpu.VMEM((CHUNK, D), jnp.float32),
        pltpu.VMEM((CHUNK, D), jnp.float32),
        pltpu.VMEM_SHARED((N_PAD, D), jnp.float32),
        pltpu.SemaphoreType.DMA,
        pltpu.SemaphoreType.DMA,
        pltpu.SemaphoreType.DMA,
        pltpu.SemaphoreType.DMA,
    ],
)(_sc_body)


def _add_body(p_ref, o_ref):
    o_ref[...] = p_ref[0] + p_ref[1]


def _combine(partials):
    blk = 1000
    return pl.pallas_call(
        _add_body,
        out_shape=jax.ShapeDtypeStruct((N_NODES, D), jnp.float32),
        grid=(N_NODES // blk,),
        in_specs=[pl.BlockSpec((NC, blk, D), lambda i: (0, i, 0))],
        out_specs=pl.BlockSpec((blk, D), lambda i: (i, 0)),
    )(partials)


def kernel(edge_index, x, edge_weight):
    ei = edge_index.astype(jnp.int32)
    src = ei[0].reshape(N_SUPER, SUPER, CHUNK)
    dst = ei[1].reshape(N_SUPER, SUPER, CHUNK)
    w = edge_weight.astype(jnp.float32).reshape(N_SUPER, SUPER, CHUNK)
    partials = _sc_call(src, dst, w, x)
    return _combine(partials)


# speedup vs baseline: 1.2616x; 1.2616x over previous
import functools

import jax
import jax.numpy as jnp
from jax import lax
from jax.experimental import pallas as pl
from jax.experimental.pallas import tpu as pltpu
from jax.experimental.pallas import tpu_sc as plsc

N_NODES = 10000
N_EDGES = 320000
D = 128
CHUNK = 128
SUPER = 20
NC = 2
NS = 16
NW = NC * NS
N_CHUNKS = N_EDGES // CHUNK
N_SUPER = N_CHUNKS // SUPER
N_PAD = 10112
ROWS_PER_TILE = N_PAD // NS
NBUF = 2


def _sc_body(pack_hbm, w_hbm, x_hbm, out_hbm,
             idx_v, w_v, rows0, rows1, acc_sh, gsem0, gsem1):
    cid = lax.axis_index("c")
    sid = lax.axis_index("s")
    wid = sid * NC + cid
    rows = (rows0, rows1)
    gsems = (gsem0, gsem1)

    def _zrow(i, _):
        for c in range(D // 16):
            rows0[i, pl.ds(c * 16, 16)] = jnp.zeros((16,), jnp.float32)
        return 0
    lax.fori_loop(0, CHUNK, _zrow, 0)
    row0 = sid * ROWS_PER_TILE
    for b in range(ROWS_PER_TILE // CHUNK):
        pltpu.sync_copy(rows0, acc_sh.at[pl.ds(row0 + b * CHUNK, CHUNK)])
    plsc.subcore_barrier()

    n_my = (N_SUPER - wid + NW - 1) // NW

    def _super(k, _):
        s = wid + k * NW
        pltpu.sync_copy(pack_hbm.at[s], idx_v)
        pltpu.sync_copy(w_hbm.at[s], w_v)

        ghandles = {}

        def _gather(j):
            ghandles[j] = pltpu.async_copy(
                x_hbm.at[idx_v.at[0, j]], rows[j % NBUF], gsems[j % NBUF])

        _gather(0)
        for j in range(SUPER):
            buf = rows[j % NBUF]
            ghandles.pop(j).wait()
            if j + 1 < SUPER:
                _gather(j + 1)

            def _scale(g, _):
                wgrp = w_v[j, pl.ds(g * 16, 16)]
                for lane in range(16):
                    ws = wgrp[lane]
                    row = g * 16 + lane
                    for c in range(D // 16):
                        seg = buf[row, pl.ds(c * 16, 16)]
                        buf[row, pl.ds(c * 16, 16)] = seg * ws
                return 0
            lax.fori_loop(0, CHUNK // 16, _scale, 0)

            pltpu.sync_copy(buf, acc_sh.at[idx_v.at[1, j]], add=True)
        return 0
    lax.fori_loop(0, n_my, _super, 0)

    plsc.subcore_barrier()
    pltpu.sync_copy(acc_sh.at[pl.ds(row0, ROWS_PER_TILE)],
                    out_hbm.at[cid, pl.ds(row0, ROWS_PER_TILE)])


_sc_call = functools.partial(
    pl.kernel,
    mesh=plsc.VectorSubcoreMesh(core_axis_name="c", subcore_axis_name="s"),
    out_type=jax.ShapeDtypeStruct((NC, N_PAD, D), jnp.float32),
    scratch_types=[
        pltpu.VMEM((2, SUPER, CHUNK), jnp.int32),
        pltpu.VMEM((SUPER, CHUNK), jnp.float32),
        pltpu.VMEM((CHUNK, D), jnp.float32),
        pltpu.VMEM((CHUNK, D), jnp.float32),
        pltpu.VMEM_SHARED((N_PAD, D), jnp.float32),
        pltpu.SemaphoreType.DMA,
        pltpu.SemaphoreType.DMA,
    ],
)(_sc_body)


def _add_body(p_ref, o_ref):
    o_ref[...] = p_ref[0] + p_ref[1]


def _combine(partials):
    blk = 1000
    return pl.pallas_call(
        _add_body,
        out_shape=jax.ShapeDtypeStruct((N_NODES, D), jnp.float32),
        grid=(N_NODES // blk,),
        in_specs=[pl.BlockSpec((NC, blk, D), lambda i: (0, i, 0))],
        out_specs=pl.BlockSpec((blk, D), lambda i: (i, 0)),
    )(partials)


def kernel(edge_index, x, edge_weight):
    ei = edge_index.astype(jnp.int32)
    src = ei[0].reshape(N_SUPER, SUPER, CHUNK)
    dst = ei[1].reshape(N_SUPER, SUPER, CHUNK)
    w = edge_weight.astype(jnp.float32).reshape(N_SUPER, SUPER, CHUNK)
    pack = jnp.stack([src, dst], axis=1)
    partials = _sc_call(pack, w, x)
    return _combine(partials)


# speedup vs baseline: 1.3097x; 1.0381x over previous
import functools

import jax
import jax.numpy as jnp
from jax import lax
from jax.experimental import pallas as pl
from jax.experimental.pallas import tpu as pltpu
from jax.experimental.pallas import tpu_sc as plsc

N_NODES = 10000
N_EDGES = 320000
D = 128
CHUNK = 128
SUPER = 20
NC = 2
NS = 16
NW = NC * NS
N_CHUNKS = N_EDGES // CHUNK
N_SUPER = N_CHUNKS // SUPER
N_PAD = 10112
ROWS_PER_TILE = N_PAD // NS
NBUF = 2


def _sc_body(src_hbm, dst_hbm, w_hbm, x_hbm, out_hbm,
             src_v, dst_v, w_v, rows0, rows1, acc_sh, gsem0, gsem1):
    cid = lax.axis_index("c")
    sid = lax.axis_index("s")
    wid = sid * NC + cid
    rows = (rows0, rows1)
    gsems = (gsem0, gsem1)

    def _zrow(i, _):
        for c in range(D // 16):
            rows0[i, pl.ds(c * 16, 16)] = jnp.zeros((16,), jnp.float32)
        return 0
    lax.fori_loop(0, CHUNK, _zrow, 0)
    row0 = sid * ROWS_PER_TILE
    for b in range(ROWS_PER_TILE // CHUNK):
        pltpu.sync_copy(rows0, acc_sh.at[pl.ds(row0 + b * CHUNK, CHUNK)])
    plsc.subcore_barrier()

    n_my = (N_SUPER - wid + NW - 1) // NW

    def _gather(j, buf, gsem):
        return pltpu.async_copy(x_hbm.at[src_v.at[j]], buf, gsem)

    def _step(j, buf, gsem):
        _gather(j, buf, gsem).wait()

    def _super(k, _):
        s = wid + k * NW
        pltpu.sync_copy(src_hbm.at[s], src_v)
        pltpu.sync_copy(dst_hbm.at[s], dst_v)
        pltpu.sync_copy(w_hbm.at[s], w_v)

        _gather(0, rows0, gsem0)

        def _chunk(j, _):
            def _body(buf, bufn, gsem, gsemn):
                pltpu.make_async_copy(
                    x_hbm.at[src_v.at[j]], buf, gsem).wait()

                @pl.when(j + 1 < SUPER)
                def _():
                    _gather(j + 1, bufn, gsemn)

                def _scale(g, _):
                    wgrp = w_v[j, pl.ds(g * 16, 16)]
                    for lane in range(16):
                        ws = wgrp[lane]
                        row = g * 16 + lane
                        for c in range(D // 16):
                            seg = buf[row, pl.ds(c * 16, 16)]
                            buf[row, pl.ds(c * 16, 16)] = seg * ws
                    return 0
                lax.fori_loop(0, CHUNK // 16, _scale, 0)

                pltpu.sync_copy(buf, acc_sh.at[dst_v.at[j]], add=True)

            @pl.when(lax.rem(j, 2) == 0)
            def _():
                _body(rows0, rows1, gsem0, gsem1)

            @pl.when(lax.rem(j, 2) == 1)
            def _():
                _body(rows1, rows0, gsem1, gsem0)
            return 0
        lax.fori_loop(0, SUPER, _chunk, 0)
        return 0
    lax.fori_loop(0, n_my, _super, 0)

    plsc.subcore_barrier()
    pltpu.sync_copy(acc_sh.at[pl.ds(row0, ROWS_PER_TILE)],
                    out_hbm.at[cid, pl.ds(row0, ROWS_PER_TILE)])


_sc_call = functools.partial(
    pl.kernel,
    mesh=plsc.VectorSubcoreMesh(core_axis_name="c", subcore_axis_name="s"),
    out_type=jax.ShapeDtypeStruct((NC, N_PAD, D), jnp.float32),
    scratch_types=[
        pltpu.VMEM((SUPER, CHUNK), jnp.int32),
        pltpu.VMEM((SUPER, CHUNK), jnp.int32),
        pltpu.VMEM((SUPER, CHUNK), jnp.float32),
        pltpu.VMEM((CHUNK, D), jnp.float32),
        pltpu.VMEM((CHUNK, D), jnp.float32),
        pltpu.VMEM_SHARED((N_PAD, D), jnp.float32),
        pltpu.SemaphoreType.DMA,
        pltpu.SemaphoreType.DMA,
    ],
)(_sc_body)


def _add_body(p_ref, o_ref):
    o_ref[...] = p_ref[0] + p_ref[1]


def _combine(partials):
    blk = 1000
    return pl.pallas_call(
        _add_body,
        out_shape=jax.ShapeDtypeStruct((N_NODES, D), jnp.float32),
        grid=(N_NODES // blk,),
        in_specs=[pl.BlockSpec((NC, blk, D), lambda i: (0, i, 0))],
        out_specs=pl.BlockSpec((blk, D), lambda i: (i, 0)),
    )(partials)


def kernel(edge_index, x, edge_weight):
    ei = edge_index.astype(jnp.int32)
    src = ei[0].reshape(N_SUPER, SUPER, CHUNK)
    dst = ei[1].reshape(N_SUPER, SUPER, CHUNK)
    w = edge_weight.astype(jnp.float32).reshape(N_SUPER, SUPER, CHUNK)
    partials = _sc_call(src, dst, w, x)
    return _combine(partials)
